# baseline (speedup 1.0000x reference)
import functools

import jax
import jax.numpy as jnp
from jax import lax
from jax.experimental import pallas as pl
from jax.experimental.pallas import tpu as pltpu
from jax.experimental.pallas import tpu_sc as plsc

D_MODEL = 768
NODE_LEN = 512
BATCH = 16
CHUNK = 64
HALF = NODE_LEN // 2
LANES = 16
VECS_PER_ROW = D_MODEL // LANES


def _mk_hidden_sc(flat, starts, ends):
    total = flat.shape[0]
    mesh = plsc.VectorSubcoreMesh(core_axis_name="c", subcore_axis_name="s")

    @functools.partial(
        pl.kernel,
        out_type=(
            jax.ShapeDtypeStruct((BATCH, NODE_LEN, D_MODEL), jnp.float32),
            jax.ShapeDtypeStruct((BATCH, NODE_LEN), jnp.int32),
        ),
        mesh=mesh,
        compiler_params=pltpu.CompilerParams(
            use_tc_tiling_on_sc=False, needs_layout_passes=False),
        scratch_types=[
            pltpu.VMEM((BATCH,), jnp.int32),
            pltpu.VMEM((BATCH,), jnp.int32),
            pltpu.VMEM((CHUNK,), jnp.int32),
            pltpu.VMEM((CHUNK, D_MODEL), jnp.float32),
            pltpu.VMEM((CHUNK, D_MODEL), jnp.float32),
            pltpu.VMEM((HALF,), jnp.int32),
            pltpu.SemaphoreType.DMA,
        ],
    )
    def k(flat_hbm, starts_hbm, ends_hbm, out_hbm, mask_hbm,
          s_v, e_v, idx_v, chunk_v, zero_v, mask_v, sem):
        wid = lax.axis_index("s") * 2 + lax.axis_index("c")
        g = wid // 2
        r0 = (wid % 2) * HALF

        iota16 = lax.iota(jnp.int32, LANES)
        pltpu.async_copy(starts_hbm, s_v, sem).wait()
        pltpu.async_copy(ends_hbm, e_v, sem).wait()
        start = jnp.max(jnp.where(iota16 == g, s_v[...], 0))
        n = jnp.max(jnp.where(iota16 == g, e_v[...], 0)) - start
        L = jnp.where(n == 1, 0, jnp.minimum(n, NODE_LEN))
        s_rel = jnp.clip(L - r0, 0, HALF)

        iota = lax.iota(jnp.int32, LANES)
        zvec = jnp.zeros((LANES,), jnp.float32)

        for v in range(HALF // LANES):
            mask_v[pl.ds(v * LANES, LANES)] = (
                (iota + (r0 + v * LANES)) < L).astype(jnp.int32)
        pltpu.sync_copy(mask_v, mask_hbm.at[g, pl.ds(r0, HALF)])

        @pl.when(s_rel < HALF)
        def _():
            @pl.loop(0, CHUNK)
            def _(j):
                for kv in range(VECS_PER_ROW):
                    zero_v[j, pl.ds(kv * LANES, LANES)] = zvec

        for c0 in range(0, HALF, CHUNK):
            full = s_rel >= (c0 + CHUNK)
            empty = s_rel <= c0
            dst = out_hbm.at[g, pl.ds(r0 + c0, CHUNK)]

            @pl.when(full)
            def _():
                pltpu.sync_copy(flat_hbm.at[pl.ds(start + r0 + c0, CHUNK)], dst)

            @pl.when(empty)
            def _():
                pltpu.sync_copy(zero_v, dst)

            @pl.when(jnp.logical_not(jnp.logical_or(full, empty)))
            def _():
                base = start + r0 + c0
                for v in range(CHUNK // LANES):
                    idx_v[pl.ds(v * LANES, LANES)] = jnp.minimum(
                        iota + (base + v * LANES), total - 1)
                pltpu.async_copy(flat_hbm.at[idx_v], chunk_v, sem).wait()
                z0 = s_rel - c0

                @pl.loop(0, CHUNK)
                def _(j):
                    @pl.when(j >= z0)
                    def _():
                        for kv in range(VECS_PER_ROW):
                            chunk_v[j, pl.ds(kv * LANES, LANES)] = zvec

                pltpu.sync_copy(chunk_v, dst)

    return k(flat, starts, ends)


def kernel(flat, cu_seqlens):
    starts = cu_seqlens[:-1]
    ends = cu_seqlens[1:]
    return _mk_hidden_sc(flat, starts, ends)


# speedup vs baseline: 4.5007x; 4.5007x over previous
import functools

import jax
import jax.numpy as jnp
from jax import lax
from jax.experimental import pallas as pl
from jax.experimental.pallas import tpu as pltpu
from jax.experimental.pallas import tpu_sc as plsc

D_MODEL = 768
NODE_LEN = 512
BATCH = 16
CHUNK = 64
HALF = NODE_LEN // 2
LANES = 16
VECS_PER_ROW = D_MODEL // LANES


def _mk_hidden_sc(flat, starts, ends):
    total = flat.shape[0]
    mesh = plsc.VectorSubcoreMesh(core_axis_name="c", subcore_axis_name="s")

    @functools.partial(
        pl.kernel,
        out_type=(
            jax.ShapeDtypeStruct((BATCH, NODE_LEN, D_MODEL), jnp.float32),
            jax.ShapeDtypeStruct((BATCH, NODE_LEN), jnp.int32),
        ),
        mesh=mesh,
        compiler_params=pltpu.CompilerParams(
            use_tc_tiling_on_sc=False, needs_layout_passes=False),
        scratch_types=[
            pltpu.VMEM((BATCH,), jnp.int32),
            pltpu.VMEM((BATCH,), jnp.int32),
            pltpu.VMEM((CHUNK,), jnp.int32),
            pltpu.VMEM((CHUNK, D_MODEL), jnp.float32),
            pltpu.VMEM((CHUNK, D_MODEL), jnp.float32),
            pltpu.VMEM((HALF,), jnp.int32),
            pltpu.SemaphoreType.DMA,
        ],
    )
    def k(flat_hbm, starts_hbm, ends_hbm, out_hbm, mask_hbm,
          s_v, e_v, idx_v, chunk_v, zero_v, mask_v, sem):
        wid = lax.axis_index("s") * 2 + lax.axis_index("c")
        g = wid // 2
        r0 = (wid % 2) * HALF

        iota16 = lax.iota(jnp.int32, LANES)
        pltpu.async_copy(starts_hbm, s_v, sem).wait()
        pltpu.async_copy(ends_hbm, e_v, sem).wait()
        start = jnp.max(jnp.where(iota16 == g, s_v[...], 0))
        n = jnp.max(jnp.where(iota16 == g, e_v[...], 0)) - start
        L = jnp.where(n == 1, 0, jnp.minimum(n, NODE_LEN))
        s_rel = jnp.clip(L - r0, 0, HALF)

        iota = lax.iota(jnp.int32, LANES)
        zvec = jnp.zeros((LANES,), jnp.float32)

        for v in range(HALF // LANES):
            mask_v[pl.ds(v * LANES, LANES)] = (
                (iota + (r0 + v * LANES)) < L).astype(jnp.int32)
        pltpu.sync_copy(mask_v, mask_hbm.at[g, pl.ds(r0, HALF)])

        @pl.when(s_rel < HALF)
        def _():
            @pl.loop(0, CHUNK)
            def _(j):
                for kv in range(VECS_PER_ROW):
                    zero_v[j, pl.ds(kv * LANES, LANES)] = zvec

        for c0 in range(0, HALF, CHUNK):
            full = s_rel >= (c0 + CHUNK)
            empty = s_rel <= c0
            dst = out_hbm.at[g, pl.ds(r0 + c0, CHUNK)]

            @pl.when(full)
            def _():
                pltpu.sync_copy(flat_hbm.at[pl.ds(start + r0 + c0, CHUNK)],
                                chunk_v)
                pltpu.sync_copy(chunk_v, dst)

            @pl.when(empty)
            def _():
                pltpu.sync_copy(zero_v, dst)

            @pl.when(jnp.logical_not(jnp.logical_or(full, empty)))
            def _():
                base = start + r0 + c0
                for v in range(CHUNK // LANES):
                    idx_v[pl.ds(v * LANES, LANES)] = jnp.minimum(
                        iota + (base + v * LANES), total - 1)
                pltpu.async_copy(flat_hbm.at[idx_v], chunk_v, sem).wait()
                z0 = s_rel - c0

                @pl.loop(0, CHUNK)
                def _(j):
                    @pl.when(j >= z0)
                    def _():
                        for kv in range(VECS_PER_ROW):
                            chunk_v[j, pl.ds(kv * LANES, LANES)] = zvec

                pltpu.sync_copy(chunk_v, dst)

    return k(flat, starts, ends)


def kernel(flat, cu_seqlens):
    starts = cu_seqlens[:-1]
    ends = cu_seqlens[1:]
    return _mk_hidden_sc(flat, starts, ends)


# speedup vs baseline: 4.7552x; 1.0566x over previous
import functools

import jax
import jax.numpy as jnp
from jax import lax
from jax.experimental import pallas as pl
from jax.experimental.pallas import tpu as pltpu
from jax.experimental.pallas import tpu_sc as plsc

D_MODEL = 768
NODE_LEN = 512
BATCH = 16
CHUNK = 32
HALF = NODE_LEN // 2
NCHUNK = HALF // CHUNK
LANES = 16
VECS_PER_ROW = D_MODEL // LANES


def _mk_hidden_sc(flat, se):
    total = flat.shape[0]
    mesh = plsc.VectorSubcoreMesh(core_axis_name="c", subcore_axis_name="s")

    @functools.partial(
        pl.kernel,
        out_type=(
            jax.ShapeDtypeStruct((BATCH, NODE_LEN, D_MODEL), jnp.float32),
            jax.ShapeDtypeStruct((BATCH, NODE_LEN), jnp.int32),
        ),
        mesh=mesh,
        compiler_params=pltpu.CompilerParams(
            use_tc_tiling_on_sc=False, needs_layout_passes=False),
        scratch_types=[
            pltpu.VMEM((2, LANES), jnp.int32),
            pltpu.VMEM((CHUNK,), jnp.int32),
            pltpu.VMEM((CHUNK, D_MODEL), jnp.float32),
            pltpu.VMEM((CHUNK, D_MODEL), jnp.float32),
            pltpu.VMEM((CHUNK, D_MODEL), jnp.float32),
            pltpu.VMEM((HALF,), jnp.int32),
            pltpu.SemaphoreType.DMA,
            pltpu.SemaphoreType.DMA,
            pltpu.SemaphoreType.DMA,
            pltpu.SemaphoreType.DMA,
        ],
    )
    def k(flat_hbm, se_hbm, out_hbm, mask_hbm,
          se_v, idx_v, buf0, buf1, zero_v, mask_v,
          sem_in, sem_out0, sem_out1, sem_mask):
        bufs = (buf0, buf1)
        sems = (sem_out0, sem_out1)
        wid = lax.axis_index("s") * 2 + lax.axis_index("c")
        g = wid // 2
        r0 = (wid % 2) * HALF

        iota16 = lax.iota(jnp.int32, LANES)
        pltpu.async_copy(se_hbm, se_v, sem_in).wait()
        start = jnp.max(jnp.where(iota16 == g, se_v[0, :], 0))
        n = jnp.max(jnp.where(iota16 == g, se_v[1, :], 0)) - start
        L = jnp.where(n == 1, 0, jnp.minimum(n, NODE_LEN))
        s_rel = jnp.clip(L - r0, 0, HALF)

        zvec = jnp.zeros((LANES,), jnp.float32)

        for v in range(HALF // LANES):
            mask_v[pl.ds(v * LANES, LANES)] = (
                (iota16 + (r0 + v * LANES)) < L).astype(jnp.int32)
        mask_copy = pltpu.async_copy(
            mask_v, mask_hbm.at[g, pl.ds(r0, HALF)], sem_mask)

        @pl.when(s_rel < HALF)
        def _():
            @pl.loop(0, CHUNK)
            def _(j):
                for kv in range(VECS_PER_ROW):
                    zero_v[j, pl.ds(kv * LANES, LANES)] = zvec

        for c in range(NCHUNK):
            c0 = c * CHUNK
            b = c % 2
            dst = out_hbm.at[g, pl.ds(r0 + c0, CHUNK)]
            if c >= 2:
                pltpu.make_async_copy(
                    bufs[b], out_hbm.at[g, pl.ds(r0 + (c - 2) * CHUNK, CHUNK)],
                    sems[b]).wait()

            full = s_rel >= (c0 + CHUNK)
            empty = s_rel <= c0

            @pl.when(full)
            def _():
                pltpu.sync_copy(flat_hbm.at[pl.ds(start + r0 + c0, CHUNK)],
                                bufs[b])
                pltpu.async_copy(bufs[b], dst, sems[b])

            @pl.when(empty)
            def _():
                pltpu.async_copy(zero_v, dst, sems[b])

            @pl.when(jnp.logical_not(jnp.logical_or(full, empty)))
            def _():
                base = start + r0 + c0
                for v in range(CHUNK // LANES):
                    idx_v[pl.ds(v * LANES, LANES)] = jnp.minimum(
                        iota16 + (base + v * LANES), total - 1)
                pltpu.async_copy(flat_hbm.at[idx_v], bufs[b], sem_in).wait()
                z0 = s_rel - c0

                @pl.loop(0, CHUNK)
                def _(j):
                    @pl.when(j >= z0)
                    def _():
                        for kv in range(VECS_PER_ROW):
                            bufs[b][j, pl.ds(kv * LANES, LANES)] = zvec

                pltpu.async_copy(bufs[b], dst, sems[b])

        for c in (NCHUNK - 2, NCHUNK - 1):
            pltpu.make_async_copy(
                bufs[c % 2], out_hbm.at[g, pl.ds(r0 + c * CHUNK, CHUNK)],
                sems[c % 2]).wait()
        mask_copy.wait()

    return k(flat, se)


def kernel(flat, cu_seqlens):
    se = jnp.stack([cu_seqlens[:-1], cu_seqlens[1:]])
    return _mk_hidden_sc(flat, se)


# speedup vs baseline: 4.7597x; 1.0009x over previous
import functools

import jax
import jax.numpy as jnp
from jax import lax
from jax.experimental import pallas as pl
from jax.experimental.pallas import tpu as pltpu
from jax.experimental.pallas import tpu_sc as plsc

D_MODEL = 768
NODE_LEN = 512
BATCH = 16
CHUNK = 32
HALF = NODE_LEN // 2
NCHUNK = HALF // CHUNK
LANES = 16
VECS_PER_ROW = D_MODEL // LANES


def _mk_hidden_sc(flat, se):
    total = flat.shape[0]
    mesh = plsc.VectorSubcoreMesh(core_axis_name="c", subcore_axis_name="s")

    @functools.partial(
        pl.kernel,
        out_type=(
            jax.ShapeDtypeStruct((BATCH, NODE_LEN, D_MODEL), jnp.float32),
            jax.ShapeDtypeStruct((BATCH, NODE_LEN), jnp.int32),
        ),
        mesh=mesh,
        compiler_params=pltpu.CompilerParams(
            use_tc_tiling_on_sc=False, needs_layout_passes=False),
        scratch_types=[
            pltpu.VMEM((2, LANES), jnp.int32),
            pltpu.VMEM((2, CHUNK), jnp.int32),
            pltpu.VMEM((CHUNK, D_MODEL), jnp.float32),
            pltpu.VMEM((CHUNK, D_MODEL), jnp.float32),
            pltpu.VMEM((CHUNK, D_MODEL), jnp.float32),
            pltpu.VMEM((HALF,), jnp.int32),
            pltpu.SemaphoreType.DMA,
            pltpu.SemaphoreType.DMA,
            pltpu.SemaphoreType.DMA,
            pltpu.SemaphoreType.DMA,
            pltpu.SemaphoreType.DMA,
        ],
    )
    def k(flat_hbm, se_hbm, out_hbm, mask_hbm,
          se_v, idx_v, buf0, buf1, zero_v, mask_v,
          sem_in0, sem_in1, sem_out0, sem_out1, sem_mask):
        bufs = (buf0, buf1)
        sems_in = (sem_in0, sem_in1)
        sems_out = (sem_out0, sem_out1)
        wid = lax.axis_index("s") * 2 + lax.axis_index("c")
        g = wid // 2
        r0 = (wid % 2) * HALF

        iota16 = lax.iota(jnp.int32, LANES)
        pltpu.async_copy(se_hbm, se_v, sem_in0).wait()
        start = jnp.max(jnp.where(iota16 == g, se_v[0, :], 0))
        n = jnp.max(jnp.where(iota16 == g, se_v[1, :], 0)) - start
        L = jnp.where(n == 1, 0, jnp.minimum(n, NODE_LEN))
        s_rel = jnp.clip(L - r0, 0, HALF)

        zvec = jnp.zeros((LANES,), jnp.float32)

        def issue_in(c):
            b = c % 2

            @pl.when(s_rel > c * CHUNK)
            def _():
                base = start + r0 + c * CHUNK
                for v in range(CHUNK // LANES):
                    idx_v[b, pl.ds(v * LANES, LANES)] = jnp.minimum(
                        iota16 + (base + v * LANES), total - 1)
                pltpu.async_copy(flat_hbm.at[idx_v.at[b]], bufs[b],
                                 sems_in[b])

        def absorb_out(c):
            pltpu.make_async_copy(
                bufs[c % 2], out_hbm.at[g, pl.ds(r0 + c * CHUNK, CHUNK)],
                sems_out[c % 2]).wait()

        for v in range(HALF // LANES):
            mask_v[pl.ds(v * LANES, LANES)] = (
                (iota16 + (r0 + v * LANES)) < L).astype(jnp.int32)
        mask_copy = pltpu.async_copy(
            mask_v, mask_hbm.at[g, pl.ds(r0, HALF)], sem_mask)

        @pl.when(s_rel < HALF)
        def _():
            @pl.loop(0, CHUNK)
            def _(j):
                for kv in range(VECS_PER_ROW):
                    zero_v[j, pl.ds(kv * LANES, LANES)] = zvec

        issue_in(0)
        for c in range(NCHUNK):
            c0 = c * CHUNK
            b = c % 2
            dst = out_hbm.at[g, pl.ds(r0 + c0, CHUNK)]
            full = s_rel >= (c0 + CHUNK)
            empty = s_rel <= c0

            @pl.when(jnp.logical_not(empty))
            def _():
                pltpu.make_async_copy(flat_hbm.at[idx_v.at[b]], bufs[b],
                                      sems_in[b]).wait()

            @pl.when(jnp.logical_not(jnp.logical_or(full, empty)))
            def _():
                z0 = s_rel - c0

                @pl.loop(0, CHUNK)
                def _(j):
                    @pl.when(j >= z0)
                    def _():
                        for kv in range(VECS_PER_ROW):
                            bufs[b][j, pl.ds(kv * LANES, LANES)] = zvec

            @pl.when(empty)
            def _():
                pltpu.async_copy(zero_v, dst, sems_out[b])

            @pl.when(jnp.logical_not(empty))
            def _():
                pltpu.async_copy(bufs[b], dst, sems_out[b])

            if c + 1 < NCHUNK:
                if c >= 1:
                    absorb_out(c - 1)
                issue_in(c + 1)

        absorb_out(NCHUNK - 2)
        absorb_out(NCHUNK - 1)
        mask_copy.wait()

    return k(flat, se)


def kernel(flat, cu_seqlens):
    se = jnp.stack([cu_seqlens[:-1], cu_seqlens[1:]])
    return _mk_hidden_sc(flat, se)


# speedup vs baseline: 8.7935x; 1.8475x over previous
import functools

import jax
import jax.numpy as jnp
from jax import lax
from jax.experimental import pallas as pl
from jax.experimental.pallas import tpu as pltpu
from jax.experimental.pallas import tpu_sc as plsc

D_MODEL = 768
NODE_LEN = 512
BATCH = 16
CHUNK = 32
HALF = NODE_LEN // 2
NCHUNK = HALF // CHUNK
LANES = 16
VECS_PER_ROW = D_MODEL // LANES


def _mk_hidden_sc(flat, se):
    total = flat.shape[0]
    mesh = plsc.VectorSubcoreMesh(core_axis_name="c", subcore_axis_name="s")

    @functools.partial(
        pl.kernel,
        out_type=(
            jax.ShapeDtypeStruct((BATCH, NODE_LEN, D_MODEL), jnp.float32),
            jax.ShapeDtypeStruct((BATCH, NODE_LEN), jnp.int32),
        ),
        mesh=mesh,
        compiler_params=pltpu.CompilerParams(needs_layout_passes=False),
        scratch_types=[
            pltpu.VMEM((2, LANES), jnp.int32),
            pltpu.VMEM((CHUNK,), jnp.int32),
            pltpu.VMEM((CHUNK,), jnp.int32),
            pltpu.VMEM((CHUNK, D_MODEL), jnp.float32),
            pltpu.VMEM((CHUNK, D_MODEL), jnp.float32),
            pltpu.VMEM((CHUNK, D_MODEL), jnp.float32),
            pltpu.VMEM((8, NODE_LEN), jnp.int32),
            pltpu.SemaphoreType.DMA,
            pltpu.SemaphoreType.DMA,
            pltpu.SemaphoreType.DMA,
            pltpu.SemaphoreType.DMA,
            pltpu.SemaphoreType.DMA,
        ],
    )
    def k(flat_hbm, se_hbm, out_hbm, mask_hbm,
          se_v, idx0, idx1, buf0, buf1, zero_v, mask_v,
          sem_in0, sem_in1, sem_out0, sem_out1, sem_mask):
        bufs = (buf0, buf1)
        idxs = (idx0, idx1)
        sems_in = (sem_in0, sem_in1)
        sems_out = (sem_out0, sem_out1)
        wid = lax.axis_index("s") * 2 + lax.axis_index("c")
        g = wid // 2
        r0 = (wid % 2) * HALF

        iota16 = lax.iota(jnp.int32, LANES)
        pltpu.async_copy(se_hbm, se_v, sem_in0).wait()
        starts_vec = se_v[0, :]
        ends_vec = se_v[1, :]
        start = jnp.max(jnp.where(iota16 == g, starts_vec, 0))
        n = jnp.max(jnp.where(iota16 == g, ends_vec, 0)) - start
        L = jnp.where(n == 1, 0, jnp.minimum(n, NODE_LEN))
        s_rel = jnp.clip(L - r0, 0, HALF)

        zvec = jnp.zeros((LANES,), jnp.float32)

        nvec = ends_vec - starts_vec
        lvec = jnp.where(nvec == 1, 0, jnp.minimum(nvec, NODE_LEN))
        mask_copy = None
        if True:
            @pl.when(wid < 2)
            def _():
                for gr in range(8):
                    gg = wid * 8 + gr
                    lg = jnp.max(jnp.where(iota16 == gg, lvec, 0))
                    for v in range(NODE_LEN // LANES):
                        mask_v[gr, pl.ds(v * LANES, LANES)] = (
                            (iota16 + v * LANES) < lg).astype(jnp.int32)
                pltpu.async_copy(
                    mask_v, mask_hbm.at[pl.ds(wid * 8, 8)], sem_mask)

        def issue_in(c):
            b = c % 2

            @pl.when(s_rel > c * CHUNK)
            def _():
                base = start + r0 + c * CHUNK
                for v in range(CHUNK // LANES):
                    idxs[b][pl.ds(v * LANES, LANES)] = jnp.minimum(
                        iota16 + (base + v * LANES), total - 1)
                pltpu.async_copy(flat_hbm.at[idxs[b]], bufs[b], sems_in[b])

        def absorb_out(c):
            pltpu.make_async_copy(
                bufs[c % 2], out_hbm.at[g, pl.ds(r0 + c * CHUNK, CHUNK)],
                sems_out[c % 2]).wait()

        @pl.when(s_rel < HALF)
        def _():
            @pl.loop(0, CHUNK)
            def _(j):
                for kv in range(VECS_PER_ROW):
                    zero_v[j, pl.ds(kv * LANES, LANES)] = zvec

        issue_in(0)
        for c in range(NCHUNK):
            c0 = c * CHUNK
            b = c % 2
            dst = out_hbm.at[g, pl.ds(r0 + c0, CHUNK)]
            full = s_rel >= (c0 + CHUNK)
            empty = s_rel <= c0

            @pl.when(jnp.logical_not(empty))
            def _():
                pltpu.make_async_copy(flat_hbm.at[idxs[b]], bufs[b],
                                      sems_in[b]).wait()

            @pl.when(jnp.logical_not(jnp.logical_or(full, empty)))
            def _():
                z0 = s_rel - c0

                @pl.loop(0, CHUNK)
                def _(j):
                    @pl.when(j >= z0)
                    def _():
                        for kv in range(VECS_PER_ROW):
                            bufs[b][j, pl.ds(kv * LANES, LANES)] = zvec

            @pl.when(empty)
            def _():
                pltpu.async_copy(zero_v, dst, sems_out[b])

            @pl.when(jnp.logical_not(empty))
            def _():
                pltpu.async_copy(bufs[b], dst, sems_out[b])

            if c + 1 < NCHUNK:
                if c >= 1:
                    absorb_out(c - 1)
                issue_in(c + 1)

        absorb_out(NCHUNK - 2)
        absorb_out(NCHUNK - 1)

        @pl.when(wid < 2)
        def _():
            pltpu.make_async_copy(
                mask_v, mask_hbm.at[pl.ds(wid * 8, 8)], sem_mask).wait()

    return k(flat, se)


def kernel(flat, cu_seqlens):
    se = jnp.stack([cu_seqlens[:-1], cu_seqlens[1:]])
    return _mk_hidden_sc(flat, se)


# speedup vs baseline: 9.2698x; 1.0542x over previous
import functools

import jax
import jax.numpy as jnp
from jax import lax
from jax.experimental import pallas as pl
from jax.experimental.pallas import tpu as pltpu
from jax.experimental.pallas import tpu_sc as plsc

D_MODEL = 768
NODE_LEN = 512
BATCH = 16
CHUNK = 64
ZROWS = 32
HALF = NODE_LEN // 2
NCHUNK = HALF // CHUNK
LANES = 16
VECS_PER_ROW = D_MODEL // LANES


def _mk_hidden_sc(flat, se):
    total = flat.shape[0]
    mesh = plsc.VectorSubcoreMesh(core_axis_name="c", subcore_axis_name="s")

    @functools.partial(
        pl.kernel,
        out_type=(
            jax.ShapeDtypeStruct((BATCH, NODE_LEN, D_MODEL), jnp.float32),
            jax.ShapeDtypeStruct((BATCH, NODE_LEN), jnp.int32),
        ),
        mesh=mesh,
        compiler_params=pltpu.CompilerParams(needs_layout_passes=False),
        scratch_types=[
            pltpu.VMEM((2, LANES), jnp.int32),
            pltpu.VMEM((CHUNK,), jnp.int32),
            pltpu.VMEM((CHUNK,), jnp.int32),
            pltpu.VMEM((CHUNK, D_MODEL), jnp.float32),
            pltpu.VMEM((CHUNK, D_MODEL), jnp.float32),
            pltpu.VMEM((ZROWS, D_MODEL), jnp.float32),
            pltpu.VMEM((8, NODE_LEN), jnp.int32),
            pltpu.SemaphoreType.DMA,
            pltpu.SemaphoreType.DMA,
            pltpu.SemaphoreType.DMA,
            pltpu.SemaphoreType.DMA,
            pltpu.SemaphoreType.DMA,
        ],
    )
    def k(flat_hbm, se_hbm, out_hbm, mask_hbm,
          se_v, idx0, idx1, buf0, buf1, zero_v, mask_v,
          sem_in0, sem_in1, sem_out0, sem_out1, sem_mask):
        bufs = (buf0, buf1)
        idxs = (idx0, idx1)
        sems_in = (sem_in0, sem_in1)
        sems_out = (sem_out0, sem_out1)
        wid = lax.axis_index("s") * 2 + lax.axis_index("c")
        g = wid // 2
        r0 = (wid % 2) * HALF

        iota16 = lax.iota(jnp.int32, LANES)
        pltpu.async_copy(se_hbm, se_v, sem_in0).wait()
        starts_vec = se_v[0, :]
        ends_vec = se_v[1, :]
        start = jnp.max(jnp.where(iota16 == g, starts_vec, 0))
        n = jnp.max(jnp.where(iota16 == g, ends_vec, 0)) - start
        L = jnp.where(n == 1, 0, jnp.minimum(n, NODE_LEN))
        s_rel = jnp.clip(L - r0, 0, HALF)

        zvec = jnp.zeros((LANES,), jnp.float32)

        def issue_in(c):
            b = c % 2

            @pl.when(s_rel > c * CHUNK)
            def _():
                base = start + r0 + c * CHUNK
                for v in range(CHUNK // LANES):
                    idxs[b][pl.ds(v * LANES, LANES)] = jnp.minimum(
                        iota16 + (base + v * LANES), total - 1)
                pltpu.async_copy(flat_hbm.at[idxs[b]], bufs[b], sems_in[b])

        def absorb_out(c):
            pltpu.make_async_copy(
                bufs[c % 2], out_hbm.at[g, pl.ds(r0 + c * CHUNK, CHUNK)],
                sems_out[c % 2]).wait()

        issue_in(0)

        nvec = ends_vec - starts_vec
        lvec = jnp.where(nvec == 1, 0, jnp.minimum(nvec, NODE_LEN))

        @pl.when(wid < 2)
        def _():
            for gr in range(8):
                gg = wid * 8 + gr
                lg = jnp.max(jnp.where(iota16 == gg, lvec, 0))
                for v in range(NODE_LEN // LANES):
                    mask_v[gr, pl.ds(v * LANES, LANES)] = (
                        (iota16 + v * LANES) < lg).astype(jnp.int32)
            pltpu.async_copy(mask_v, mask_hbm.at[pl.ds(wid * 8, 8)], sem_mask)

        @pl.when(s_rel < HALF)
        def _():
            @pl.loop(0, ZROWS)
            def _(j):
                for kv in range(VECS_PER_ROW):
                    zero_v[j, pl.ds(kv * LANES, LANES)] = zvec

        for c in range(NCHUNK):
            c0 = c * CHUNK
            b = c % 2
            full = s_rel >= (c0 + CHUNK)
            empty = s_rel <= c0

            @pl.when(jnp.logical_not(empty))
            def _():
                pltpu.make_async_copy(flat_hbm.at[idxs[b]], bufs[b],
                                      sems_in[b]).wait()

            @pl.when(jnp.logical_not(jnp.logical_or(full, empty)))
            def _():
                z0 = s_rel - c0

                @pl.loop(0, CHUNK)
                def _(j):
                    @pl.when(j >= z0)
                    def _():
                        for kv in range(VECS_PER_ROW):
                            bufs[b][j, pl.ds(kv * LANES, LANES)] = zvec

            @pl.when(empty)
            def _():
                pltpu.async_copy(
                    zero_v, out_hbm.at[g, pl.ds(r0 + c0, ZROWS)], sems_out[b])
                pltpu.async_copy(
                    zero_v, out_hbm.at[g, pl.ds(r0 + c0 + ZROWS, ZROWS)],
                    sems_out[b])

            @pl.when(jnp.logical_not(empty))
            def _():
                pltpu.async_copy(
                    bufs[b], out_hbm.at[g, pl.ds(r0 + c0, CHUNK)], sems_out[b])

            if c + 1 < NCHUNK:
                if c >= 1:
                    absorb_out(c - 1)
                issue_in(c + 1)

        absorb_out(NCHUNK - 2)
        absorb_out(NCHUNK - 1)

        @pl.when(wid < 2)
        def _():
            pltpu.make_async_copy(
                mask_v, mask_hbm.at[pl.ds(wid * 8, 8)], sem_mask).wait()

    return k(flat, se)


def kernel(flat, cu_seqlens):
    se = jnp.stack([cu_seqlens[:-1], cu_seqlens[1:]])
    return _mk_hidden_sc(flat, se)


# speedup vs baseline: 9.2842x; 1.0016x over previous
import functools

import jax
import jax.numpy as jnp
from jax import lax
from jax.experimental import pallas as pl
from jax.experimental.pallas import tpu as pltpu
from jax.experimental.pallas import tpu_sc as plsc

D_MODEL = 768
NODE_LEN = 512
BATCH = 16
CHUNK = 64
ZROWS = 16
HALF = NODE_LEN // 2
NCHUNK = HALF // CHUNK
LANES = 16
VECS_PER_ROW = D_MODEL // LANES


def _mk_hidden_sc(flat, cu):
    total = flat.shape[0]
    mesh = plsc.VectorSubcoreMesh(core_axis_name="c", subcore_axis_name="s")

    @functools.partial(
        pl.kernel,
        out_type=(
            jax.ShapeDtypeStruct((BATCH, NODE_LEN, D_MODEL), jnp.float32),
            jax.ShapeDtypeStruct((BATCH, NODE_LEN), jnp.int32),
        ),
        mesh=mesh,
        compiler_params=pltpu.CompilerParams(needs_layout_passes=False),
        scratch_types=[
            pltpu.VMEM((LANES,), jnp.int32),
            pltpu.VMEM((LANES,), jnp.int32),
            pltpu.VMEM((CHUNK,), jnp.int32),
            pltpu.VMEM((CHUNK,), jnp.int32),
            pltpu.VMEM((CHUNK, D_MODEL), jnp.float32),
            pltpu.VMEM((CHUNK, D_MODEL), jnp.float32),
            pltpu.VMEM((ZROWS, D_MODEL), jnp.float32),
            pltpu.VMEM((8, NODE_LEN), jnp.int32),
            pltpu.SemaphoreType.DMA,
            pltpu.SemaphoreType.DMA,
            pltpu.SemaphoreType.DMA,
            pltpu.SemaphoreType.DMA,
            pltpu.SemaphoreType.DMA,
        ],
    )
    def k(flat_hbm, cu_hbm, out_hbm, mask_hbm,
          lo_v, hi_v, idx0, idx1, buf0, buf1, zero_v, mask_v,
          sem_in0, sem_in1, sem_out0, sem_out1, sem_mask):
        bufs = (buf0, buf1)
        idxs = (idx0, idx1)
        sems_in = (sem_in0, sem_in1)
        sems_out = (sem_out0, sem_out1)
        wid = lax.axis_index("s") * 2 + lax.axis_index("c")
        g = wid // 2
        r0 = (wid % 2) * HALF

        iota16 = lax.iota(jnp.int32, LANES)
        pltpu.async_copy(cu_hbm.at[pl.ds(0, LANES)], lo_v, sem_in0)
        cp_hi = pltpu.async_copy(cu_hbm.at[pl.ds(8, 9)],
                                 hi_v.at[pl.ds(0, 9)], sem_in1)
        pltpu.make_async_copy(cu_hbm.at[pl.ds(0, LANES)], lo_v,
                              sem_in0).wait()
        cp_hi.wait()
        lo = lo_v[...]
        hi = hi_v[...]

        def cu_at(i):
            from_lo = jnp.max(jnp.where(iota16 == i, lo, 0))
            from_hi = jnp.max(jnp.where(iota16 == i - 8, hi, 0))
            return jnp.where(i < LANES, from_lo, from_hi)

        def graph_len(gg):
            s = cu_at(gg)
            nn = cu_at(gg + 1) - s
            return s, jnp.where(nn == 1, 0, jnp.minimum(nn, NODE_LEN))

        start, L = graph_len(g)
        s_rel = jnp.clip(L - r0, 0, HALF)

        zvec = jnp.zeros((LANES,), jnp.float32)

        def issue_in(c):
            b = c % 2

            @pl.when(s_rel > c * CHUNK)
            def _():
                base = start + r0 + c * CHUNK
                for v in range(CHUNK // LANES):
                    idxs[b][pl.ds(v * LANES, LANES)] = jnp.minimum(
                        iota16 + (base + v * LANES), total - 1)
                pltpu.async_copy(flat_hbm.at[idxs[b]], bufs[b], sems_in[b])

        def absorb_out(c):
            pltpu.make_async_copy(
                bufs[c % 2], out_hbm.at[g, pl.ds(r0 + c * CHUNK, CHUNK)],
                sems_out[c % 2]).wait()

        issue_in(0)

        @pl.when(wid < 2)
        def _():
            for gr in range(8):
                _, lg = graph_len(wid * 8 + gr)
                for v in range(NODE_LEN // LANES):
                    mask_v[gr, pl.ds(v * LANES, LANES)] = (
                        (iota16 + v * LANES) < lg).astype(jnp.int32)
            pltpu.async_copy(mask_v, mask_hbm.at[pl.ds(wid * 8, 8)], sem_mask)

        @pl.when(s_rel < HALF)
        def _():
            @pl.loop(0, ZROWS)
            def _(j):
                for kv in range(VECS_PER_ROW):
                    zero_v[j, pl.ds(kv * LANES, LANES)] = zvec

        for c in range(NCHUNK):
            c0 = c * CHUNK
            b = c % 2
            full = s_rel >= (c0 + CHUNK)
            empty = s_rel <= c0

            @pl.when(jnp.logical_not(empty))
            def _():
                pltpu.make_async_copy(flat_hbm.at[idxs[b]], bufs[b],
                                      sems_in[b]).wait()

            @pl.when(jnp.logical_not(jnp.logical_or(full, empty)))
            def _():
                z0 = s_rel - c0

                @pl.loop(0, CHUNK)
                def _(j):
                    @pl.when(j >= z0)
                    def _():
                        for kv in range(VECS_PER_ROW):
                            bufs[b][j, pl.ds(kv * LANES, LANES)] = zvec

            @pl.when(empty)
            def _():
                for z in range(CHUNK // ZROWS):
                    pltpu.async_copy(
                        zero_v,
                        out_hbm.at[g, pl.ds(r0 + c0 + z * ZROWS, ZROWS)],
                        sems_out[b])

            @pl.when(jnp.logical_not(empty))
            def _():
                pltpu.async_copy(
                    bufs[b], out_hbm.at[g, pl.ds(r0 + c0, CHUNK)], sems_out[b])

            if c + 1 < NCHUNK:
                if c >= 1:
                    absorb_out(c - 1)
                issue_in(c + 1)

        absorb_out(NCHUNK - 2)
        absorb_out(NCHUNK - 1)

        @pl.when(wid < 2)
        def _():
            pltpu.make_async_copy(
                mask_v, mask_hbm.at[pl.ds(wid * 8, 8)], sem_mask).wait()

    return k(flat, cu)


def kernel(flat, cu_seqlens):
    return _mk_hidden_sc(flat, cu_seqlens)


# speedup vs baseline: 9.7392x; 1.0490x over previous
import functools

import jax
import jax.numpy as jnp
from jax import lax
from jax.experimental import pallas as pl
from jax.experimental.pallas import tpu as pltpu
from jax.experimental.pallas import tpu_sc as plsc

D_MODEL = 768
NODE_LEN = 512
BATCH = 16
CHUNK = 32
ZROWS = 16
RING = 4
HALF = NODE_LEN // 2
NCHUNK = HALF // CHUNK
LANES = 16
VECS_PER_ROW = D_MODEL // LANES


def _mk_hidden_sc(flat, cu):
    total = flat.shape[0]
    mesh = plsc.VectorSubcoreMesh(core_axis_name="c", subcore_axis_name="s")

    @functools.partial(
        pl.kernel,
        out_type=(
            jax.ShapeDtypeStruct((BATCH, NODE_LEN, D_MODEL), jnp.float32),
            jax.ShapeDtypeStruct((BATCH, NODE_LEN), jnp.int32),
        ),
        mesh=mesh,
        compiler_params=pltpu.CompilerParams(needs_layout_passes=False),
        scratch_types=[
            pltpu.VMEM((LANES,), jnp.int32),
            pltpu.VMEM((LANES,), jnp.int32),
            pltpu.VMEM((RING, CHUNK), jnp.int32),
            pltpu.VMEM((CHUNK, D_MODEL), jnp.float32),
            pltpu.VMEM((CHUNK, D_MODEL), jnp.float32),
            pltpu.VMEM((CHUNK, D_MODEL), jnp.float32),
            pltpu.VMEM((CHUNK, D_MODEL), jnp.float32),
            pltpu.VMEM((ZROWS, D_MODEL), jnp.float32),
            pltpu.VMEM((8, NODE_LEN), jnp.int32),
            pltpu.SemaphoreType.DMA,
            pltpu.SemaphoreType.DMA,
            pltpu.SemaphoreType.DMA,
            pltpu.SemaphoreType.DMA,
            pltpu.SemaphoreType.DMA,
            pltpu.SemaphoreType.DMA,
            pltpu.SemaphoreType.DMA,
            pltpu.SemaphoreType.DMA,
            pltpu.SemaphoreType.DMA,
        ],
    )
    def k(flat_hbm, cu_hbm, out_hbm, mask_hbm,
          lo_v, hi_v, idx_v, buf0, buf1, buf2, buf3, zero_v, mask_v,
          si0, si1, si2, si3, so0, so1, so2, so3, sem_mask):
        bufs = (buf0, buf1, buf2, buf3)
        sems_in = (si0, si1, si2, si3)
        sems_out = (so0, so1, so2, so3)
        wid = lax.axis_index("s") * 2 + lax.axis_index("c")
        g = wid // 2
        r0 = (wid % 2) * HALF

        iota16 = lax.iota(jnp.int32, LANES)
        pltpu.async_copy(cu_hbm.at[pl.ds(0, LANES)], lo_v, si0)
        cp_hi = pltpu.async_copy(cu_hbm.at[pl.ds(8, 9)],
                                 hi_v.at[pl.ds(0, 9)], si1)
        pltpu.make_async_copy(cu_hbm.at[pl.ds(0, LANES)], lo_v, si0).wait()
        cp_hi.wait()
        lo = lo_v[...]
        hi = hi_v[...]

        def cu_at(i):
            from_lo = jnp.max(jnp.where(iota16 == i, lo, 0))
            from_hi = jnp.max(jnp.where(iota16 == i - 8, hi, 0))
            return jnp.where(i < LANES, from_lo, from_hi)

        def graph_len(gg):
            s = cu_at(gg)
            nn = cu_at(gg + 1) - s
            return s, jnp.where(nn == 1, 0, jnp.minimum(nn, NODE_LEN))

        start, L = graph_len(g)
        s_rel = jnp.clip(L - r0, 0, HALF)

        zvec = jnp.zeros((LANES,), jnp.float32)

        def issue_in(c):
            b = c % RING

            @pl.when(s_rel > c * CHUNK)
            def _():
                base = start + r0 + c * CHUNK
                for v in range(CHUNK // LANES):
                    idx_v[b, pl.ds(v * LANES, LANES)] = jnp.minimum(
                        iota16 + (base + v * LANES), total - 1)
                pltpu.async_copy(flat_hbm.at[idx_v.at[b]], bufs[b],
                                 sems_in[b])

        def absorb_out(c):
            pltpu.make_async_copy(
                bufs[c % RING], out_hbm.at[g, pl.ds(r0 + c * CHUNK, CHUNK)],
                sems_out[c % RING]).wait()

        issue_in(0)
        issue_in(1)
        issue_in(2)

        @pl.when(wid < 2)
        def _():
            for gr in range(8):
                _, lg = graph_len(wid * 8 + gr)
                for v in range(NODE_LEN // LANES):
                    mask_v[gr, pl.ds(v * LANES, LANES)] = (
                        (iota16 + v * LANES) < lg).astype(jnp.int32)
            pltpu.async_copy(mask_v, mask_hbm.at[pl.ds(wid * 8, 8)], sem_mask)

        @pl.when(s_rel < HALF)
        def _():
            @pl.loop(0, ZROWS)
            def _(j):
                for kv in range(VECS_PER_ROW):
                    zero_v[j, pl.ds(kv * LANES, LANES)] = zvec

        for c in range(NCHUNK):
            c0 = c * CHUNK
            b = c % RING
            full = s_rel >= (c0 + CHUNK)
            empty = s_rel <= c0

            @pl.when(jnp.logical_not(empty))
            def _():
                pltpu.make_async_copy(flat_hbm.at[idx_v.at[b]], bufs[b],
                                      sems_in[b]).wait()

            @pl.when(jnp.logical_not(jnp.logical_or(full, empty)))
            def _():
                z0 = s_rel - c0

                @pl.loop(0, CHUNK)
                def _(j):
                    @pl.when(j >= z0)
                    def _():
                        for kv in range(VECS_PER_ROW):
                            bufs[b][j, pl.ds(kv * LANES, LANES)] = zvec

            @pl.when(empty)
            def _():
                for z in range(CHUNK // ZROWS):
                    pltpu.async_copy(
                        zero_v,
                        out_hbm.at[g, pl.ds(r0 + c0 + z * ZROWS, ZROWS)],
                        sems_out[b])

            @pl.when(jnp.logical_not(empty))
            def _():
                pltpu.async_copy(
                    bufs[b], out_hbm.at[g, pl.ds(r0 + c0, CHUNK)], sems_out[b])

            if c + RING - 1 < NCHUNK:
                if c >= 1:
                    absorb_out(c - 1)
                issue_in(c + RING - 1)

        for c in range(NCHUNK - RING, NCHUNK):
            absorb_out(c)

        @pl.when(wid < 2)
        def _():
            pltpu.make_async_copy(
                mask_v, mask_hbm.at[pl.ds(wid * 8, 8)], sem_mask).wait()

    return k(flat, cu)


def kernel(flat, cu_seqlens):
    return _mk_hidden_sc(flat, cu_seqlens)
